# baseline (speedup 1.0000x reference)
import functools

import jax
import jax.numpy as jnp
from jax import lax
from jax.experimental import pallas as pl
from jax.experimental.pallas import tpu as pltpu
from jax.experimental.pallas import tpu_sc as plsc

_BR = 32
_NS = 4
_CT = 74752
_NW = 32
_L = 16
_UNROLL = 6


def _tc_body(cw, *refs):
    x_refs = refs[:_NS]
    tgt_ref, m1_ref, xt_ref = refs[_NS], refs[_NS + 1], refs[_NS + 2]
    tgt = tgt_ref[...]
    m1 = None
    xt = None
    for k in range(_NS):
        xb = x_refs[k][...]
        col = jax.lax.broadcasted_iota(jnp.int32, xb.shape, 1) + k * cw
        is_tgt = col == tgt
        t = jnp.log(jnp.exp(xb) * 0.5) - xb
        mk = jnp.min(jnp.where(is_tgt, jnp.inf, t), axis=1, keepdims=True)
        xk = jnp.sum(jnp.where(is_tgt, xb, 0.0), axis=1, keepdims=True)
        m1 = mk if m1 is None else jnp.minimum(m1, mk)
        xt = xk if xt is None else xt + xk
    m1_ref[...] = m1
    xt_ref[...] = xt


def _tc_pass(x, tgt):
    Bn = x.shape[0]
    cw = _CT // _NS
    grid = (Bn // _BR,)
    in_specs = [
        pl.BlockSpec((_BR, cw), functools.partial(lambda k, i: (i, k), k))
        for k in range(_NS)
    ]
    in_specs.append(pl.BlockSpec((_BR, 1), lambda i: (i, 0)))
    return pl.pallas_call(
        functools.partial(_tc_body, cw),
        grid=grid,
        in_specs=in_specs,
        out_specs=[pl.BlockSpec((_BR, 1), lambda i: (i, 0))] * 2,
        out_shape=[jax.ShapeDtypeStruct((Bn, 1), x.dtype)] * 2,
    )(*([x] * _NS), tgt)


def _sc_row(slab, t16, nv):
    g0 = lax.iota(jnp.int32, _L) + _CT
    inf16 = jnp.full((_L,), jnp.inf, jnp.float32)
    zero16 = jnp.zeros((_L,), jnp.float32)

    def step(i, carry):
        macc, xacc, gbase = carry
        us = []
        for s in range(_UNROLL):
            v = slab[pl.ds((i * _UNROLL + s) * _L, _L)]
            us.append(((jnp.exp(v) * 0.5) * jnp.exp(-v), v, gbase + s * _L))
        for u, v, g in us:
            xacc = jnp.where(g == t16, v, xacc)
        m = us[0][0]
        for u, v, g in us[1:]:
            m = jnp.minimum(m, u)
        return jnp.minimum(macc, m), xacc, gbase + _UNROLL * _L

    macc, xacc, _ = lax.fori_loop(0, nv // _UNROLL, step, (inf16, zero16, g0))
    return macc, xacc


def _sc_body(wsc, x, tgtsp, m_out, xt_out, slab0, slab1, tgts_v, outm_v,
             outx_v, sem0, sem1):
    bpw = tgtsp.shape[0] // _NW
    wid = lax.axis_index("s") * 2 + lax.axis_index("c")
    base = wid * bpw
    pltpu.sync_copy(tgtsp.at[pl.ds(base, bpw)], tgts_v)
    nv = wsc // _L
    slabs = [slab0, slab1]
    sems = [sem0, sem1]
    handles = [None, None]
    handles[0] = pltpu.async_copy(x.at[base, pl.ds(_CT, wsc)], slab0, sem0)
    for j in range(bpw):
        b = j % 2
        if j + 1 < bpw:
            handles[1 - b] = pltpu.async_copy(
                x.at[base + j + 1, pl.ds(_CT, wsc)], slabs[1 - b], sems[1 - b])
        handles[b].wait()
        t16 = tgts_v.at[j][...]
        macc, xacc = _sc_row(slabs[b], t16, nv)
        outm_v[pl.ds(j * _L, _L)] = macc
        outx_v[pl.ds(j * _L, _L)] = xacc
    pltpu.sync_copy(outm_v, m_out.at[pl.ds(base * _L, bpw * _L)])
    pltpu.sync_copy(outx_v, xt_out.at[pl.ds(base * _L, bpw * _L)])


def _sc_pass(x, tgtsp):
    Bn, Cn = x.shape
    wsc = Cn - _CT
    bpw = Bn // _NW
    mesh = plsc.VectorSubcoreMesh(core_axis_name="c", subcore_axis_name="s")
    return pl.kernel(
        functools.partial(_sc_body, wsc),
        mesh=mesh,
        out_type=[jax.ShapeDtypeStruct((Bn * _L,), jnp.float32)] * 2,
        scratch_types=[
            pltpu.VMEM((wsc,), jnp.float32),
            pltpu.VMEM((wsc,), jnp.float32),
            pltpu.VMEM((bpw, _L), jnp.int32),
            pltpu.VMEM((bpw * _L,), jnp.float32),
            pltpu.VMEM((bpw * _L,), jnp.float32),
            pltpu.SemaphoreType.DMA,
            pltpu.SemaphoreType.DMA,
        ],
    )(x, tgtsp)


def _combine_body(m1_ref, xt_ref, tgt_ref, msc_ref, xsc_ref, out_ref):
    m_sc = jnp.min(msc_ref[...], axis=1, keepdims=True)
    m1 = jnp.minimum(m1_ref[...], jnp.log(m_sc))
    xsc = jnp.sum(xsc_ref[...], axis=1, keepdims=True)
    xt = jnp.where(tgt_ref[...] >= _CT, xsc, xt_ref[...])
    d2 = jnp.log((jnp.exp(xt) + 1.0) * 0.5)
    out_ref[...] = -(m1 + d2)


def kernel(x, target):
    Bn, Cn = x.shape
    tgt = target.astype(jnp.int32).reshape(Bn, 1)
    tgtsp = jnp.broadcast_to(tgt, (Bn, _L))
    m1_tc, xt_tc = _tc_pass(x, tgt)
    m_sc, xt_sc = _sc_pass(x, tgtsp)
    return pl.pallas_call(
        _combine_body,
        out_shape=jax.ShapeDtypeStruct((Bn, 1), x.dtype),
    )(m1_tc, xt_tc, tgt, m_sc.reshape(Bn, _L), xt_sc.reshape(Bn, _L))


# speedup vs baseline: 1.0061x; 1.0061x over previous
import functools

import jax
import jax.numpy as jnp
from jax import lax
from jax.experimental import pallas as pl
from jax.experimental.pallas import tpu as pltpu
from jax.experimental.pallas import tpu_sc as plsc

_BR = 32
_NS = 4
_CT = 70144
_NW = 32
_L = 16
_UNROLL = 6


def _tc_body(cw, *refs):
    x_refs = refs[:_NS]
    tgt_ref, m1_ref, xt_ref = refs[_NS], refs[_NS + 1], refs[_NS + 2]
    tgt = tgt_ref[...]
    m1 = None
    xt = None
    for k in range(_NS):
        xb = x_refs[k][...]
        col = jax.lax.broadcasted_iota(jnp.int32, xb.shape, 1) + k * cw
        is_tgt = col == tgt
        t = jnp.log(jnp.exp(xb) * 0.5) - xb
        mk = jnp.min(jnp.where(is_tgt, jnp.inf, t), axis=1, keepdims=True)
        xk = jnp.sum(jnp.where(is_tgt, xb, 0.0), axis=1, keepdims=True)
        m1 = mk if m1 is None else jnp.minimum(m1, mk)
        xt = xk if xt is None else xt + xk
    m1_ref[...] = m1
    xt_ref[...] = xt


def _tc_pass(x, tgt):
    Bn = x.shape[0]
    cw = _CT // _NS
    grid = (Bn // _BR,)
    in_specs = [
        pl.BlockSpec((_BR, cw), functools.partial(lambda k, i: (i, k), k))
        for k in range(_NS)
    ]
    in_specs.append(pl.BlockSpec((_BR, 1), lambda i: (i, 0)))
    return pl.pallas_call(
        functools.partial(_tc_body, cw),
        grid=grid,
        in_specs=in_specs,
        out_specs=[pl.BlockSpec((_BR, 1), lambda i: (i, 0))] * 2,
        out_shape=[jax.ShapeDtypeStruct((Bn, 1), x.dtype)] * 2,
    )(*([x] * _NS), tgt)


def _sc_row(slab, t16, nv):
    g0 = lax.iota(jnp.int32, _L) + _CT
    inf16 = jnp.full((_L,), jnp.inf, jnp.float32)
    zero16 = jnp.zeros((_L,), jnp.float32)

    def step(i, carry):
        macc, xacc, gbase = carry
        us = []
        for s in range(_UNROLL):
            v = slab[pl.ds((i * _UNROLL + s) * _L, _L)]
            us.append(((jnp.exp(v) * 0.5) * jnp.exp(-v), v, gbase + s * _L))
        for u, v, g in us:
            xacc = jnp.where(g == t16, v, xacc)
        m = us[0][0]
        for u, v, g in us[1:]:
            m = jnp.minimum(m, u)
        return jnp.minimum(macc, m), xacc, gbase + _UNROLL * _L

    macc, xacc, _ = lax.fori_loop(0, nv // _UNROLL, step, (inf16, zero16, g0))
    return macc, xacc


def _sc_body(wsc, x, tgtsp, m_out, xt_out, slab0, slab1, tgts_v, outm_v,
             outx_v, sem0, sem1):
    bpw = tgtsp.shape[0] // _NW
    wid = lax.axis_index("s") * 2 + lax.axis_index("c")
    base = wid * bpw
    pltpu.sync_copy(tgtsp.at[pl.ds(base, bpw)], tgts_v)
    nv = wsc // _L
    slabs = [slab0, slab1]
    sems = [sem0, sem1]
    handles = [None, None]
    handles[0] = pltpu.async_copy(x.at[base, pl.ds(_CT, wsc)], slab0, sem0)
    for j in range(bpw):
        b = j % 2
        if j + 1 < bpw:
            handles[1 - b] = pltpu.async_copy(
                x.at[base + j + 1, pl.ds(_CT, wsc)], slabs[1 - b], sems[1 - b])
        handles[b].wait()
        t16 = tgts_v.at[j][...]
        macc, xacc = _sc_row(slabs[b], t16, nv)
        outm_v[pl.ds(j * _L, _L)] = macc
        outx_v[pl.ds(j * _L, _L)] = xacc
    pltpu.sync_copy(outm_v, m_out.at[pl.ds(base * _L, bpw * _L)])
    pltpu.sync_copy(outx_v, xt_out.at[pl.ds(base * _L, bpw * _L)])


def _sc_pass(x, tgtsp):
    Bn, Cn = x.shape
    wsc = Cn - _CT
    bpw = Bn // _NW
    mesh = plsc.VectorSubcoreMesh(core_axis_name="c", subcore_axis_name="s")
    return pl.kernel(
        functools.partial(_sc_body, wsc),
        mesh=mesh,
        out_type=[jax.ShapeDtypeStruct((Bn * _L,), jnp.float32)] * 2,
        scratch_types=[
            pltpu.VMEM((wsc,), jnp.float32),
            pltpu.VMEM((wsc,), jnp.float32),
            pltpu.VMEM((bpw, _L), jnp.int32),
            pltpu.VMEM((bpw * _L,), jnp.float32),
            pltpu.VMEM((bpw * _L,), jnp.float32),
            pltpu.SemaphoreType.DMA,
            pltpu.SemaphoreType.DMA,
        ],
    )(x, tgtsp)


def _combine_body(m1_ref, xt_ref, tgt_ref, msc_ref, xsc_ref, out_ref):
    m_sc = jnp.min(msc_ref[...], axis=1, keepdims=True)
    m1 = jnp.minimum(m1_ref[...], jnp.log(m_sc))
    xsc = jnp.sum(xsc_ref[...], axis=1, keepdims=True)
    xt = jnp.where(tgt_ref[...] >= _CT, xsc, xt_ref[...])
    d2 = jnp.log((jnp.exp(xt) + 1.0) * 0.5)
    out_ref[...] = -(m1 + d2)


def kernel(x, target):
    Bn, Cn = x.shape
    tgt = target.astype(jnp.int32).reshape(Bn, 1)
    tgtsp = jnp.broadcast_to(tgt, (Bn, _L))
    m1_tc, xt_tc = _tc_pass(x, tgt)
    m_sc, xt_sc = _sc_pass(x, tgtsp)
    return pl.pallas_call(
        _combine_body,
        out_shape=jax.ShapeDtypeStruct((Bn, 1), x.dtype),
    )(m1_tc, xt_tc, tgt, m_sc.reshape(Bn, _L), xt_sc.reshape(Bn, _L))


# speedup vs baseline: 1.0100x; 1.0039x over previous
import functools

import jax
import jax.numpy as jnp
from jax import lax
from jax.experimental import pallas as pl
from jax.experimental.pallas import tpu as pltpu
from jax.experimental.pallas import tpu_sc as plsc

_BR = 32
_NS = 4
_CT = 71680
_NW = 32
_L = 16
_UNROLL = 6


def _tc_body(cw, *refs):
    x_refs = refs[:_NS]
    tgt_ref, m1_ref, xt_ref = refs[_NS], refs[_NS + 1], refs[_NS + 2]
    tgt = tgt_ref[...]
    m1 = None
    xt = None
    for k in range(_NS):
        xb = x_refs[k][...]
        col = jax.lax.broadcasted_iota(jnp.int32, xb.shape, 1) + k * cw
        is_tgt = col == tgt
        t = jnp.log(jnp.exp(xb) * 0.5) - xb
        mk = jnp.min(jnp.where(is_tgt, jnp.inf, t), axis=1, keepdims=True)
        xk = jnp.sum(jnp.where(is_tgt, xb, 0.0), axis=1, keepdims=True)
        m1 = mk if m1 is None else jnp.minimum(m1, mk)
        xt = xk if xt is None else xt + xk
    m1_ref[...] = m1
    xt_ref[...] = xt


def _tc_pass(x, tgt):
    Bn = x.shape[0]
    cw = _CT // _NS
    grid = (Bn // _BR,)
    in_specs = [
        pl.BlockSpec((_BR, cw), functools.partial(lambda k, i: (i, k), k))
        for k in range(_NS)
    ]
    in_specs.append(pl.BlockSpec((_BR, 1), lambda i: (i, 0)))
    return pl.pallas_call(
        functools.partial(_tc_body, cw),
        grid=grid,
        in_specs=in_specs,
        out_specs=[pl.BlockSpec((_BR, 1), lambda i: (i, 0))] * 2,
        out_shape=[jax.ShapeDtypeStruct((Bn, 1), x.dtype)] * 2,
    )(*([x] * _NS), tgt)


def _sc_row(slab, t16, nv):
    g0 = lax.iota(jnp.int32, _L) + _CT
    inf16 = jnp.full((_L,), jnp.inf, jnp.float32)
    zero16 = jnp.zeros((_L,), jnp.float32)

    def step(i, carry):
        macc, xacc, gbase = carry
        us = []
        for s in range(_UNROLL):
            v = slab[pl.ds((i * _UNROLL + s) * _L, _L)]
            us.append(((jnp.exp(v) * 0.5) * jnp.exp(-v), v, gbase + s * _L))
        for u, v, g in us:
            xacc = jnp.where(g == t16, v, xacc)
        m = us[0][0]
        for u, v, g in us[1:]:
            m = jnp.minimum(m, u)
        return jnp.minimum(macc, m), xacc, gbase + _UNROLL * _L

    macc, xacc, _ = lax.fori_loop(0, nv // _UNROLL, step, (inf16, zero16, g0))
    return macc, xacc


def _sc_body(wsc, x, tgtsp, m_out, xt_out, slab0, slab1, tgts_v, outm_v,
             outx_v, sem0, sem1):
    bpw = tgtsp.shape[0] // _NW
    wid = lax.axis_index("s") * 2 + lax.axis_index("c")
    base = wid * bpw
    pltpu.sync_copy(tgtsp.at[pl.ds(base, bpw)], tgts_v)
    nv = wsc // _L
    slabs = [slab0, slab1]
    sems = [sem0, sem1]
    handles = [None, None]
    handles[0] = pltpu.async_copy(x.at[base, pl.ds(_CT, wsc)], slab0, sem0)
    for j in range(bpw):
        b = j % 2
        if j + 1 < bpw:
            handles[1 - b] = pltpu.async_copy(
                x.at[base + j + 1, pl.ds(_CT, wsc)], slabs[1 - b], sems[1 - b])
        handles[b].wait()
        t16 = tgts_v.at[j][...]
        macc, xacc = _sc_row(slabs[b], t16, nv)
        outm_v[pl.ds(j * _L, _L)] = macc
        outx_v[pl.ds(j * _L, _L)] = xacc
    pltpu.sync_copy(outm_v, m_out.at[pl.ds(base * _L, bpw * _L)])
    pltpu.sync_copy(outx_v, xt_out.at[pl.ds(base * _L, bpw * _L)])


def _sc_pass(x, tgtsp):
    Bn, Cn = x.shape
    wsc = Cn - _CT
    bpw = Bn // _NW
    mesh = plsc.VectorSubcoreMesh(core_axis_name="c", subcore_axis_name="s")
    return pl.kernel(
        functools.partial(_sc_body, wsc),
        mesh=mesh,
        out_type=[jax.ShapeDtypeStruct((Bn * _L,), jnp.float32)] * 2,
        scratch_types=[
            pltpu.VMEM((wsc,), jnp.float32),
            pltpu.VMEM((wsc,), jnp.float32),
            pltpu.VMEM((bpw, _L), jnp.int32),
            pltpu.VMEM((bpw * _L,), jnp.float32),
            pltpu.VMEM((bpw * _L,), jnp.float32),
            pltpu.SemaphoreType.DMA,
            pltpu.SemaphoreType.DMA,
        ],
    )(x, tgtsp)


def _combine_body(m1_ref, xt_ref, tgt_ref, msc_ref, xsc_ref, out_ref):
    m_sc = jnp.min(msc_ref[...], axis=1, keepdims=True)
    m1 = jnp.minimum(m1_ref[...], jnp.log(m_sc))
    xsc = jnp.sum(xsc_ref[...], axis=1, keepdims=True)
    xt = jnp.where(tgt_ref[...] >= _CT, xsc, xt_ref[...])
    d2 = jnp.log((jnp.exp(xt) + 1.0) * 0.5)
    out_ref[...] = -(m1 + d2)


def kernel(x, target):
    Bn, Cn = x.shape
    tgt = target.astype(jnp.int32).reshape(Bn, 1)
    tgtsp = jnp.broadcast_to(tgt, (Bn, _L))
    m1_tc, xt_tc = _tc_pass(x, tgt)
    m_sc, xt_sc = _sc_pass(x, tgtsp)
    return pl.pallas_call(
        _combine_body,
        out_shape=jax.ShapeDtypeStruct((Bn, 1), x.dtype),
    )(m1_tc, xt_tc, tgt, m_sc.reshape(Bn, _L), xt_sc.reshape(Bn, _L))
